# baseline (speedup 1.0000x reference)
import jax
import jax.numpy as jnp
from jax import lax
from jax.experimental import pallas as pl
from jax.experimental.pallas import tpu as pltpu
from jax.experimental.pallas import tpu_sc as plsc

NU = 6000
NI = 4000
NN = NU + NI
NNP = 10240
NPAD = NNP - NN
NE = 320000
D = 128
ALPHA = 0.1
BETA = 0.9

NC = 2
NS = 16
EB = 128
NQ = 8
QB = 20
NBLK = NQ * QB
EPS = NBLK * EB
NEP = NS * EPS
PAD_SRC = 6015
PAD_DST = 4064

RU = 6016
RI = 4096
BN = RU + RI
UPS = RU // NS
IPS = RI // NS
DEGN = 6144
DEGPS = DEGN // NS

_mesh = plsc.VectorSubcoreMesh(
    core_axis_name="c", subcore_axis_name="s", num_cores=NC, num_subcores=NS
)


def _deg_body(edges_hbm, ones_hbm, zeros_hbm, out_hbm, idx_v, ones_v, deg_sh):
    c = lax.axis_index("c")
    s = lax.axis_index("s")
    pltpu.sync_copy(zeros_hbm.at[pl.ds(s * DEGPS, DEGPS)],
                    deg_sh.at[pl.ds(s * DEGPS, DEGPS)])
    pltpu.sync_copy(ones_hbm, ones_v)
    pltpu.sync_copy(edges_hbm.at[s], idx_v)
    plsc.subcore_barrier()

    def blk(j, carry):
        pltpu.sync_copy(ones_v, deg_sh.at[idx_v.at[c, j]], add=True)
        return carry

    lax.fori_loop(0, NBLK, blk, 0)
    plsc.subcore_barrier()
    pltpu.sync_copy(deg_sh.at[pl.ds(s * DEGPS, DEGPS)],
                    out_hbm.at[c, pl.ds(s * DEGPS, DEGPS)])


_deg_call = pl.kernel(
    _deg_body,
    out_type=jax.ShapeDtypeStruct((NC, DEGN), jnp.float32),
    mesh=_mesh,
    scratch_types=[
        pltpu.VMEM((2, NBLK, EB), jnp.int32),
        pltpu.VMEM((EB,), jnp.float32),
        pltpu.VMEM_SHARED((DEGN,), jnp.float32),
    ],
)


def _prop_body(edges_hbm, table_hbm, zeros_hbm, out_hbm, idx_v, buf, sem_a,
               sem_b, sem_i, b_sh):
    c = lax.axis_index("c")
    s = lax.axis_index("s")

    @pl.when(c == 0)
    def _c0():
        pltpu.sync_copy(zeros_hbm.at[pl.ds(s * UPS, UPS)],
                        b_sh.at[pl.ds(s * UPS, UPS)])
        pltpu.sync_copy(table_hbm.at[pl.ds(NU + s * IPS, IPS)],
                        b_sh.at[pl.ds(RU + s * IPS, IPS)])

    @pl.when(c == 1)
    def _c1():
        pltpu.sync_copy(table_hbm.at[pl.ds(s * UPS, UPS)],
                        b_sh.at[pl.ds(s * UPS, UPS)])
        pltpu.sync_copy(zeros_hbm.at[pl.ds(s * IPS, IPS)],
                        b_sh.at[pl.ds(RU + s * IPS, IPS)])

    plsc.subcore_barrier()

    buf_a = buf.at[0]
    buf_b = buf.at[1]

    pltpu.sync_copy(edges_hbm.at[c, s, 0], idx_v.at[0])
    pltpu.async_copy(edges_hbm.at[c, s, 1], idx_v.at[1], sem_i)

    def chunk(e, carry):
        r = lax.rem(e, 2)
        idx_e = idx_v.at[r]

        @pl.when(e > 0)
        def _wait_prefetch():
            pltpu.make_async_copy(edges_hbm.at[c, s, e], idx_e, sem_i).wait()

        pltpu.async_copy(b_sh.at[idx_e.at[0, 0]], buf_a, sem_a)

        @pl.when(e < NQ - 1)
        def _prefetch_next():
            pltpu.async_copy(edges_hbm.at[c, s, e + 1],
                             idx_v.at[lax.rem(e + 1, 2)], sem_i)

        def pair(p, cc):
            ja = 2 * p
            jb = 2 * p + 1
            pltpu.async_copy(b_sh.at[idx_e.at[0, jb]], buf_b, sem_b)
            pltpu.make_async_copy(b_sh.at[idx_e.at[0, ja]], buf_a, sem_a).wait()
            pltpu.sync_copy(buf_a, b_sh.at[idx_e.at[1, ja]], add=True)

            @pl.when(p < QB // 2 - 1)
            def _next():
                pltpu.async_copy(b_sh.at[idx_e.at[0, ja + 2]], buf_a, sem_a)

            pltpu.make_async_copy(b_sh.at[idx_e.at[0, jb]], buf_b, sem_b).wait()
            pltpu.sync_copy(buf_b, b_sh.at[idx_e.at[1, jb]], add=True)
            return cc

        lax.fori_loop(0, QB // 2, pair, 0)
        return carry

    lax.fori_loop(0, NQ, chunk, 0)
    plsc.subcore_barrier()

    @pl.when(c == 0)
    def _w0():
        pltpu.sync_copy(b_sh.at[pl.ds(s * UPS, UPS)],
                        out_hbm.at[c, pl.ds(s * UPS, UPS)])

    @pl.when(c == 1)
    def _w1():
        pltpu.sync_copy(b_sh.at[pl.ds(RU + s * IPS, IPS)],
                        out_hbm.at[c, pl.ds(s * IPS, IPS)])


_prop_call = pl.kernel(
    _prop_body,
    out_type=jax.ShapeDtypeStruct((NC, RU, D), jnp.float32),
    mesh=_mesh,
    scratch_types=[
        pltpu.VMEM((2, 2, QB, EB), jnp.int32),
        pltpu.VMEM((2, EB, D), jnp.float32),
        pltpu.SemaphoreType.DMA,
        pltpu.SemaphoreType.DMA,
        pltpu.SemaphoreType.DMA,
        pltpu.VMEM_SHARED((BN, D), jnp.float32),
    ],
)


RB = 1024
GRID = NNP // RB


def _prep_body(deg_ref, h0_ref, dinv_ref, hbar_ref):
    dinv = lax.rsqrt(jnp.maximum(deg_ref[...], 1.0))
    dinv_ref[...] = dinv
    hbar_ref[...] = h0_ref[...] * dinv


_prep_call = pl.pallas_call(
    _prep_body,
    grid=(GRID,),
    in_specs=[
        pl.BlockSpec((RB, 1), lambda i: (i, 0)),
        pl.BlockSpec((RB, D), lambda i: (i, 0)),
    ],
    out_specs=[
        pl.BlockSpec((RB, 1), lambda i: (i, 0)),
        pl.BlockSpec((RB, D), lambda i: (i, 0)),
    ],
    out_shape=[
        jax.ShapeDtypeStruct((NNP, 1), jnp.float32),
        jax.ShapeDtypeStruct((NNP, D), jnp.float32),
    ],
)


def _upd1_body(raw_ref, h0_ref, dinv_ref, accout_ref, hbar_ref):
    dinv = dinv_ref[...]
    h = ALPHA * h0_ref[...] + BETA * (raw_ref[...] * dinv)
    accout_ref[...] = h0_ref[...] + h
    hbar_ref[...] = h * dinv


def _upd2_body(raw_ref, h0_ref, acc_ref, dinv_ref, fin_ref):
    dinv = dinv_ref[...]
    h = ALPHA * h0_ref[...] + BETA * (raw_ref[...] * dinv)
    fin_ref[...] = (acc_ref[...] + h) * (1.0 / 3.0)


_spec_d = pl.BlockSpec((RB, D), lambda i: (i, 0))
_spec_1 = pl.BlockSpec((RB, 1), lambda i: (i, 0))

_upd1_call = pl.pallas_call(
    _upd1_body,
    grid=(GRID,),
    in_specs=[_spec_d, _spec_d, _spec_1],
    out_specs=[_spec_d, _spec_d],
    out_shape=[
        jax.ShapeDtypeStruct((NNP, D), jnp.float32),
        jax.ShapeDtypeStruct((NNP, D), jnp.float32),
    ],
)

_upd2_call = pl.pallas_call(
    _upd2_body,
    grid=(GRID,),
    in_specs=[_spec_d, _spec_d, _spec_d, _spec_1],
    out_specs=_spec_d,
    out_shape=jax.ShapeDtypeStruct((NNP, D), jnp.float32),
)


def kernel(g, user_embeddings, item_v_feat, item_t_feat, item_embeddings,
           W_t, b_t, gamma_t, beta_t, a_t, W_v, b_v, gamma_v, beta_v, a_v):
    src = jnp.concatenate(
        [g[0].astype(jnp.int32), jnp.full((NEP - NE,), PAD_SRC, jnp.int32)])
    dst = jnp.concatenate(
        [g[1].astype(jnp.int32), jnp.full((NEP - NE,), PAD_DST, jnp.int32)])
    edges_deg = jnp.concatenate(
        [src.reshape(NS, 1, NBLK, EB), dst.reshape(NS, 1, NBLK, EB)], axis=1
    )
    src_q = src.reshape(NS, 1, NQ, 1, QB, EB)
    dst_q = dst.reshape(NS, 1, NQ, 1, QB, EB)
    core0 = jnp.concatenate([dst_q + RU, src_q], axis=3)
    core1 = jnp.concatenate([src_q, dst_q + RU], axis=3)
    edges_prop = jnp.concatenate([core0, core1], axis=1).transpose(1, 0, 2, 3, 4, 5)

    h0 = jnp.concatenate(
        [user_embeddings, item_embeddings,
         jnp.zeros((NPAD, D), jnp.float32)], axis=0)
    ones_eb = jnp.ones((EB,), jnp.float32)
    zeros_deg = jnp.zeros((DEGN,), jnp.float32)
    zeros_tab = jnp.zeros((RU, D), jnp.float32)

    deg_parts = _deg_call(edges_deg, ones_eb, zeros_deg)
    deg = jnp.concatenate(
        [deg_parts[0, :NU, None], deg_parts[1, :NI, None],
         jnp.zeros((NPAD, 1), jnp.float32)], axis=0)
    dinv, hbar = _prep_call(deg, h0)

    def raw_of(parts):
        return jnp.concatenate(
            [parts[0, :NU], parts[1, :NI],
             jnp.zeros((NPAD, D), jnp.float32)], axis=0)

    parts = _prop_call(edges_prop, hbar, zeros_tab)
    acc, hbar = _upd1_call(raw_of(parts), h0, dinv)
    parts = _prop_call(edges_prop, hbar, zeros_tab)
    fin = _upd2_call(raw_of(parts), h0, acc, dinv)
    return fin[:NN]


# speedup vs baseline: 1.0363x; 1.0363x over previous
import jax
import jax.numpy as jnp
from jax import lax
from jax.experimental import pallas as pl
from jax.experimental.pallas import tpu as pltpu
from jax.experimental.pallas import tpu_sc as plsc

NU = 6000
NI = 4000
NN = NU + NI
NNP = 10240
NPAD = NNP - NN
NE = 320000
D = 128
ALPHA = 0.1
BETA = 0.9

NC = 2
NS = 16
EB = 128
NQ = 4
QB = 40
NBLK = NQ * QB
EPS = NBLK * EB
NEP = NS * EPS
PAD_SRC = 6015
PAD_DST = 4064

RU = 6016
RI = 4096
BN = RU + RI
UPS = RU // NS
IPS = RI // NS
DEGN = 6144
DEGPS = DEGN // NS

_mesh = plsc.VectorSubcoreMesh(
    core_axis_name="c", subcore_axis_name="s", num_cores=NC, num_subcores=NS
)


def _deg_body(edges_hbm, ones_hbm, zeros_hbm, out_hbm, idx_v, ones_v, deg_sh):
    c = lax.axis_index("c")
    s = lax.axis_index("s")
    pltpu.sync_copy(zeros_hbm.at[pl.ds(s * DEGPS, DEGPS)],
                    deg_sh.at[pl.ds(s * DEGPS, DEGPS)])
    pltpu.sync_copy(ones_hbm, ones_v)
    pltpu.sync_copy(edges_hbm.at[s], idx_v)
    plsc.subcore_barrier()

    def blk(j, carry):
        pltpu.sync_copy(ones_v, deg_sh.at[idx_v.at[c, j]], add=True)
        return carry

    lax.fori_loop(0, NBLK, blk, 0)
    plsc.subcore_barrier()
    pltpu.sync_copy(deg_sh.at[pl.ds(s * DEGPS, DEGPS)],
                    out_hbm.at[c, pl.ds(s * DEGPS, DEGPS)])


_deg_call = pl.kernel(
    _deg_body,
    out_type=jax.ShapeDtypeStruct((NC, DEGN), jnp.float32),
    mesh=_mesh,
    scratch_types=[
        pltpu.VMEM((2, NBLK, EB), jnp.int32),
        pltpu.VMEM((EB,), jnp.float32),
        pltpu.VMEM_SHARED((DEGN,), jnp.float32),
    ],
)


def _prop_body(edges_hbm, table_hbm, zeros_hbm, out_hbm, idx_v, buf, sem_a,
               sem_b, b_sh):
    c = lax.axis_index("c")
    s = lax.axis_index("s")

    @pl.when(c == 0)
    def _c0():
        pltpu.sync_copy(zeros_hbm.at[pl.ds(s * UPS, UPS)],
                        b_sh.at[pl.ds(s * UPS, UPS)])
        pltpu.sync_copy(table_hbm.at[pl.ds(NU + s * IPS, IPS)],
                        b_sh.at[pl.ds(RU + s * IPS, IPS)])

    @pl.when(c == 1)
    def _c1():
        pltpu.sync_copy(table_hbm.at[pl.ds(s * UPS, UPS)],
                        b_sh.at[pl.ds(s * UPS, UPS)])
        pltpu.sync_copy(zeros_hbm.at[pl.ds(s * IPS, IPS)],
                        b_sh.at[pl.ds(RU + s * IPS, IPS)])

    plsc.subcore_barrier()

    buf_a = buf.at[0]
    buf_b = buf.at[1]

    def quarter(q, carry):
        pltpu.sync_copy(edges_hbm.at[c, s, q], idx_v)
        pltpu.async_copy(b_sh.at[idx_v.at[0, 0]], buf_a, sem_a)

        def pair(p, cc):
            ja = 2 * p
            jb = 2 * p + 1
            pltpu.async_copy(b_sh.at[idx_v.at[0, jb]], buf_b, sem_b)
            pltpu.make_async_copy(b_sh.at[idx_v.at[0, ja]], buf_a, sem_a).wait()
            pltpu.sync_copy(buf_a, b_sh.at[idx_v.at[1, ja]], add=True)

            @pl.when(p < QB // 2 - 1)
            def _next():
                pltpu.async_copy(b_sh.at[idx_v.at[0, ja + 2]], buf_a, sem_a)

            pltpu.make_async_copy(b_sh.at[idx_v.at[0, jb]], buf_b, sem_b).wait()
            pltpu.sync_copy(buf_b, b_sh.at[idx_v.at[1, jb]], add=True)
            return cc

        lax.fori_loop(0, QB // 2, pair, 0)
        return carry

    lax.fori_loop(0, NQ, quarter, 0)
    plsc.subcore_barrier()

    @pl.when(c == 0)
    def _w0():
        pltpu.sync_copy(b_sh.at[pl.ds(s * UPS, UPS)],
                        out_hbm.at[c, pl.ds(s * UPS, UPS)])

    @pl.when(c == 1)
    def _w1():
        pltpu.sync_copy(b_sh.at[pl.ds(RU + s * IPS, IPS)],
                        out_hbm.at[c, pl.ds(s * IPS, IPS)])


_prop_call = pl.kernel(
    _prop_body,
    out_type=jax.ShapeDtypeStruct((NC, RU, D), jnp.float32),
    mesh=_mesh,
    scratch_types=[
        pltpu.VMEM((2, QB, EB), jnp.int32),
        pltpu.VMEM((2, EB, D), jnp.float32),
        pltpu.SemaphoreType.DMA,
        pltpu.SemaphoreType.DMA,
        pltpu.VMEM_SHARED((BN, D), jnp.float32),
    ],
)


RB = 1024
GRID = NNP // RB


def _prep_body(deg_ref, h0_ref, dinv_ref, hbar_ref):
    dinv = lax.rsqrt(jnp.maximum(deg_ref[...], 1.0))
    dinv_ref[...] = dinv
    hbar_ref[...] = h0_ref[...] * dinv


_prep_call = pl.pallas_call(
    _prep_body,
    grid=(GRID,),
    in_specs=[
        pl.BlockSpec((RB, 1), lambda i: (i, 0)),
        pl.BlockSpec((RB, D), lambda i: (i, 0)),
    ],
    out_specs=[
        pl.BlockSpec((RB, 1), lambda i: (i, 0)),
        pl.BlockSpec((RB, D), lambda i: (i, 0)),
    ],
    out_shape=[
        jax.ShapeDtypeStruct((NNP, 1), jnp.float32),
        jax.ShapeDtypeStruct((NNP, D), jnp.float32),
    ],
)


def _upd1_body(raw_ref, h0_ref, dinv_ref, accout_ref, hbar_ref):
    dinv = dinv_ref[...]
    h = ALPHA * h0_ref[...] + BETA * (raw_ref[...] * dinv)
    accout_ref[...] = h0_ref[...] + h
    hbar_ref[...] = h * dinv


def _upd2_body(raw_ref, h0_ref, acc_ref, dinv_ref, fin_ref):
    dinv = dinv_ref[...]
    h = ALPHA * h0_ref[...] + BETA * (raw_ref[...] * dinv)
    fin_ref[...] = (acc_ref[...] + h) * (1.0 / 3.0)


_spec_d = pl.BlockSpec((RB, D), lambda i: (i, 0))
_spec_1 = pl.BlockSpec((RB, 1), lambda i: (i, 0))

_upd1_call = pl.pallas_call(
    _upd1_body,
    grid=(GRID,),
    in_specs=[_spec_d, _spec_d, _spec_1],
    out_specs=[_spec_d, _spec_d],
    out_shape=[
        jax.ShapeDtypeStruct((NNP, D), jnp.float32),
        jax.ShapeDtypeStruct((NNP, D), jnp.float32),
    ],
)

_upd2_call = pl.pallas_call(
    _upd2_body,
    grid=(GRID,),
    in_specs=[_spec_d, _spec_d, _spec_d, _spec_1],
    out_specs=_spec_d,
    out_shape=jax.ShapeDtypeStruct((NNP, D), jnp.float32),
)


def kernel(g, user_embeddings, item_v_feat, item_t_feat, item_embeddings,
           W_t, b_t, gamma_t, beta_t, a_t, W_v, b_v, gamma_v, beta_v, a_v):
    src = jnp.concatenate(
        [g[0].astype(jnp.int32), jnp.full((NEP - NE,), PAD_SRC, jnp.int32)])
    dst = jnp.concatenate(
        [g[1].astype(jnp.int32), jnp.full((NEP - NE,), PAD_DST, jnp.int32)])
    edges_deg = jnp.concatenate(
        [src.reshape(NS, 1, NBLK, EB), dst.reshape(NS, 1, NBLK, EB)], axis=1
    )
    src_q = src.reshape(NS, 1, NQ, 1, QB, EB)
    dst_q = dst.reshape(NS, 1, NQ, 1, QB, EB)
    core0 = jnp.concatenate([dst_q + RU, src_q], axis=3)
    core1 = jnp.concatenate([src_q, dst_q + RU], axis=3)
    edges_prop = jnp.concatenate([core0, core1], axis=1).transpose(1, 0, 2, 3, 4, 5)

    h0 = jnp.concatenate(
        [user_embeddings, item_embeddings,
         jnp.zeros((NPAD, D), jnp.float32)], axis=0)
    ones_eb = jnp.ones((EB,), jnp.float32)
    zeros_deg = jnp.zeros((DEGN,), jnp.float32)
    zeros_tab = jnp.zeros((RU, D), jnp.float32)

    deg_parts = _deg_call(edges_deg, ones_eb, zeros_deg)
    deg = jnp.concatenate(
        [deg_parts[0, :NU, None], deg_parts[1, :NI, None],
         jnp.zeros((NPAD, 1), jnp.float32)], axis=0)
    dinv, hbar = _prep_call(deg, h0)

    def raw_of(parts):
        return jnp.concatenate(
            [parts[0, :NU], parts[1, :NI],
             jnp.zeros((NPAD, D), jnp.float32)], axis=0)

    parts = _prop_call(edges_prop, hbar, zeros_tab)
    acc, hbar = _upd1_call(raw_of(parts), h0, dinv)
    parts = _prop_call(edges_prop, hbar, zeros_tab)
    fin = _upd2_call(raw_of(parts), h0, acc, dinv)
    return fin[:NN]
